# baseline (speedup 1.0000x reference)
import functools
import jax
import jax.numpy as jnp
from jax import lax
from jax.experimental import pallas as pl
from jax.experimental.pallas import tpu as pltpu
from jax.experimental.pallas import tpu_sc as plsc

NUM_CLASSES = 100000
DIM = 64
BATCH = 16384
NC = 2
NS = 16
NW = NC * NS
ROWS_PER_W = BATCH // NW
CHUNK = 128
NCHUNK = ROWS_PER_W // CHUNK
LANES = 16


def _sc_body(y_hbm, df_hbm, ct_hbm, out_hbm, idx_v, df_v, ct_v, acc_v, sem):
    wid = lax.axis_index("s") * NC + lax.axis_index("c")
    base = wid * ROWS_PER_W
    for j in range(NCHUNK):
        pltpu.sync_copy(y_hbm.at[pl.ds(base + j * CHUNK, CHUNK)], idx_v.at[j])
    copies = [
        pltpu.async_copy(ct_hbm.at[idx_v.at[j]], ct_v.at[j], sem)
        for j in range(NCHUNK)
    ]
    copies.append(
        pltpu.async_copy(df_hbm.at[pl.ds(base, ROWS_PER_W)], df_v, sem))
    for c in copies:
        c.wait()

    def row_body(i, acc):
        for j in range(NCHUNK):
            for c in range(DIM // LANES):
                d = (df_v[j * CHUNK + i, pl.ds(c * LANES, LANES)]
                     - ct_v[j, i, pl.ds(c * LANES, LANES)])
                acc = acc + d * d
        return acc

    acc_v[...] = lax.fori_loop(0, CHUNK, row_body,
                               jnp.zeros((LANES,), jnp.float32))
    pltpu.sync_copy(acc_v, out_hbm.at[wid])


_sc_call = pl.kernel(
    _sc_body,
    out_type=jax.ShapeDtypeStruct((NW, LANES), jnp.float32),
    mesh=plsc.VectorSubcoreMesh(core_axis_name="c", subcore_axis_name="s"),
    compiler_params=pltpu.CompilerParams(use_tc_tiling_on_sc=False),
    scratch_types=[
        pltpu.VMEM((NCHUNK, CHUNK), jnp.int32),
        pltpu.VMEM((ROWS_PER_W, DIM), jnp.float32),
        pltpu.VMEM((NCHUNK, CHUNK, DIM), jnp.float32),
        pltpu.VMEM((LANES,), jnp.float32),
        pltpu.SemaphoreType.DMA,
    ],
)


@jax.jit
def kernel(y, deep_feat, centers):
    partials = _sc_call(y.astype(jnp.int32), deep_feat, centers)
    return 0.5 * jnp.sqrt(jnp.sum(partials)) / BATCH


# speedup vs baseline: 1.0018x; 1.0018x over previous
import functools
import jax
import jax.numpy as jnp
from jax import lax
from jax.experimental import pallas as pl
from jax.experimental.pallas import tpu as pltpu
from jax.experimental.pallas import tpu_sc as plsc

NUM_CLASSES = 100000
DIM = 64
BATCH = 16384
NC = 2
NS = 16
NW = NC * NS
ROWS_PER_W = BATCH // NW
CHUNK = 128
NCHUNK = ROWS_PER_W // CHUNK
LANES = 16


def _sc_body(y_hbm, df_hbm, ct_hbm, out_hbm, idx_v, df_v, ct_v, acc_v, sem):
    wid = lax.axis_index("s") * NC + lax.axis_index("c")
    base = wid * ROWS_PER_W
    for j in range(NCHUNK):
        pltpu.sync_copy(y_hbm.at[pl.ds(base + j * CHUNK, CHUNK)], idx_v.at[j])
    copies = [
        pltpu.async_copy(ct_hbm.at[idx_v.at[j]], ct_v.at[j], sem)
        for j in range(NCHUNK)
    ]
    copies.append(
        pltpu.async_copy(df_hbm.at[pl.ds(base * DIM, ROWS_PER_W * DIM)],
                         df_v, sem))
    for c in copies:
        c.wait()

    def row_body(i, acc):
        for j in range(NCHUNK):
            for c in range(DIM // LANES):
                d = (df_v[pl.ds((j * CHUNK + i) * DIM + c * LANES, LANES)]
                     - ct_v[j, i, pl.ds(c * LANES, LANES)])
                acc = acc + d * d
        return acc

    acc_v[...] = lax.fori_loop(0, CHUNK, row_body,
                               jnp.zeros((LANES,), jnp.float32))
    pltpu.sync_copy(acc_v, out_hbm.at[wid])


_sc_call = pl.kernel(
    _sc_body,
    out_type=jax.ShapeDtypeStruct((NW, LANES), jnp.float32),
    mesh=plsc.VectorSubcoreMesh(core_axis_name="c", subcore_axis_name="s"),
    compiler_params=pltpu.CompilerParams(use_tc_tiling_on_sc=False),
    scratch_types=[
        pltpu.VMEM((NCHUNK, CHUNK), jnp.int32),
        pltpu.VMEM((ROWS_PER_W * DIM,), jnp.float32),
        pltpu.VMEM((NCHUNK, CHUNK, DIM), jnp.float32),
        pltpu.VMEM((LANES,), jnp.float32),
        pltpu.SemaphoreType.DMA,
    ],
)


@jax.jit
def kernel(y, deep_feat, centers):
    partials = _sc_call(y.astype(jnp.int32), deep_feat.reshape(-1), centers)
    return 0.5 * jnp.sqrt(jnp.sum(partials)) / BATCH


# speedup vs baseline: 1.3432x; 1.3408x over previous
import functools
import jax
import jax.numpy as jnp
from jax import lax
from jax.experimental import pallas as pl
from jax.experimental.pallas import tpu as pltpu
from jax.experimental.pallas import tpu_sc as plsc

NUM_CLASSES = 100000
DIM = 64
BATCH = 16384
NC = 2
NS = 16
NW = NC * NS
ROWS_PER_W = BATCH // NW
CHUNK = 64
NCHUNK = ROWS_PER_W // CHUNK
LANES = 16


def _sc_body(y_hbm, df_hbm, ct_hbm, out_hbm, idx_v, df_v, ct_v,
             acc_v, gsem, dsem):
    wid = lax.axis_index("s") * NC + lax.axis_index("c")
    base = wid * ROWS_PER_W
    pltpu.sync_copy(y_hbm.at[pl.ds(base, ROWS_PER_W)], idx_v)

    def issue(c, b):
        pltpu.async_copy(df_hbm.at[pl.ds(base + c * CHUNK, CHUNK)],
                         df_v.at[b], dsem)

        def issue_group(g, _):
            vec = idx_v[pl.ds(c * CHUNK + g * LANES, LANES)]
            for i in range(LANES):
                pltpu.async_copy(ct_hbm.at[pl.ds(vec[i], 1)],
                                 ct_v.at[b, pl.ds(g * LANES + i, 1)], gsem)
            return 0

        lax.fori_loop(0, CHUNK // LANES, issue_group, 0)

    def drain(b):
        pltpu.make_async_copy(ct_hbm.at[pl.ds(0, CHUNK)], ct_v.at[b],
                              gsem).wait()
        pltpu.make_async_copy(df_hbm.at[pl.ds(0, CHUNK)], df_v.at[b],
                              dsem).wait()

    def compute(b, acc):
        def row_body(i, a):
            for cc in range(DIM // LANES):
                d = (df_v[b, i, pl.ds(cc * LANES, LANES)]
                     - ct_v[b, i, pl.ds(cc * LANES, LANES)])
                a = a + d * d
            return a

        return lax.fori_loop(0, CHUNK, row_body, acc)

    acc = jnp.zeros((LANES,), jnp.float32)
    issue(0, 0)
    for c in range(NCHUNK):
        b = c % 2
        if c + 1 < NCHUNK:
            issue(c + 1, 1 - b)
        drain(b)
        acc = compute(b, acc)

    acc_v[...] = acc
    pltpu.sync_copy(acc_v, out_hbm.at[wid])


_sc_call = pl.kernel(
    _sc_body,
    out_type=jax.ShapeDtypeStruct((NW, LANES), jnp.float32),
    mesh=plsc.VectorSubcoreMesh(core_axis_name="c", subcore_axis_name="s"),
    compiler_params=pltpu.CompilerParams(use_tc_tiling_on_sc=True),
    scratch_types=[
        pltpu.VMEM((ROWS_PER_W,), jnp.int32),
        pltpu.VMEM((2, CHUNK, DIM), jnp.float32),
        pltpu.VMEM((2, CHUNK, DIM), jnp.float32),
        pltpu.VMEM((LANES,), jnp.float32),
        pltpu.SemaphoreType.DMA,
        pltpu.SemaphoreType.DMA,
    ],
)


@jax.jit
def kernel(y, deep_feat, centers):
    partials = _sc_call(y.astype(jnp.int32), deep_feat, centers)
    return 0.5 * jnp.sqrt(jnp.sum(partials)) / BATCH
